# Initial kernel scaffold; baseline (speedup 1.0000x reference)
import jax
import jax.numpy as jnp
from jax.experimental import pallas as pl


def kernel(link_idx, time_idx, link_table, time_table, cross_table, W1, b1, W2, b2, W3, b3):
    raise NotImplementedError("write your pallas kernel here")


# speedup vs baseline: 1.2610x; 1.2610x over previous
import functools

import jax
import jax.numpy as jnp
from jax import lax
from jax.experimental import pallas as pl
from jax.experimental.pallas import tpu as pltpu
from jax.experimental.pallas import tpu_sc as plsc

_N_TIMES = 288
_B = 16384
_D_LINK = 32
_D_TIME = 8

_NC = 2
_NS = 16
_NW = _NC * _NS
_CHUNK = _B // _NW
_L = 16

_sc_mesh = plsc.VectorSubcoreMesh(core_axis_name="c", subcore_axis_name="s")


@functools.partial(
    pl.kernel,
    mesh=_sc_mesh,
    compiler_params=pltpu.CompilerParams(
        use_tc_tiling_on_sc=False, needs_layout_passes=False),
    out_type=[
        jax.ShapeDtypeStruct((_B, _D_LINK), jnp.float32),
        jax.ShapeDtypeStruct((_B, _D_TIME), jnp.float32),
        jax.ShapeDtypeStruct((_B,), jnp.float32),
    ],
    scratch_types=[
        pltpu.VMEM((_CHUNK,), jnp.int32),
        pltpu.VMEM((_CHUNK,), jnp.int32),
        pltpu.VMEM((_CHUNK,), jnp.int32),
        pltpu.VMEM((_CHUNK,), jnp.int32),
        pltpu.VMEM((_CHUNK, _D_LINK), jnp.float32),
        pltpu.VMEM((_CHUNK, _D_TIME), jnp.float32),
        pltpu.VMEM((_CHUNK, 128), jnp.float32),
        pltpu.VMEM((_CHUNK,), jnp.float32),
        pltpu.SemaphoreType.DMA,
        pltpu.SemaphoreType.DMA,
        pltpu.SemaphoreType.DMA,
    ],
)
def _sc_gather(link_idx_hbm, time_idx_hbm, link_tab_hbm, time_tab_hbm,
               cross_tab_hbm, le_out, te_out, cr_out,
               li_v, ti_v, row_v, col_v, le_v, te_v, crrows_v, cr_v,
               sem_l, sem_t, sem_c):
    wid = lax.axis_index("s") * _NC + lax.axis_index("c")
    base = wid * _CHUNK
    pltpu.sync_copy(link_idx_hbm.at[pl.ds(base, _CHUNK)], li_v)
    pltpu.sync_copy(time_idx_hbm.at[pl.ds(base, _CHUNK)], ti_v)

    for k in range(_CHUNK // _L):
        a = li_v[pl.ds(k * _L, _L)]
        b = ti_v[pl.ds(k * _L, _L)]
        ci = a * _N_TIMES + b
        row_v[pl.ds(k * _L, _L)] = lax.shift_right_logical(ci, 7)
        col_v[pl.ds(k * _L, _L)] = lax.bitwise_and(ci, 127)

    cl = pltpu.async_copy(link_tab_hbm.at[li_v], le_v, sem_l)
    ct = pltpu.async_copy(time_tab_hbm.at[ti_v], te_v, sem_t)
    cc = pltpu.async_copy(cross_tab_hbm.at[row_v], crrows_v, sem_c)
    cl.wait()
    ct.wait()
    cc.wait()

    for k in range(_CHUNK // _L):
        rid = lax.broadcasted_iota(jnp.int32, (_L,), 0) + k * _L
        cid = col_v[pl.ds(k * _L, _L)]
        cr_v[pl.ds(k * _L, _L)] = plsc.load_gather(crrows_v, [rid, cid])

    pltpu.sync_copy(le_v, le_out.at[pl.ds(base, _CHUNK)])
    pltpu.sync_copy(te_v, te_out.at[pl.ds(base, _CHUNK)])
    pltpu.sync_copy(cr_v, cr_out.at[pl.ds(base, _CHUNK)])


_BLK = 2048


def _mlp_body(le_ref, te_ref, cr_ref, w1a_ref, w1b_ref, b1_ref, w2_ref,
              b2_ref, w3_ref, b3_ref, out_ref):
    dot = functools.partial(
        lax.dot_general,
        dimension_numbers=(((1,), (0,)), ((), ())),
        precision=lax.Precision.HIGHEST,
    )
    h = dot(le_ref[...], w1a_ref[...]) + dot(te_ref[...], w1b_ref[...])
    h = jnp.maximum(h + b1_ref[...], 0.0)
    h = jnp.maximum(dot(h, w2_ref[...]) + b2_ref[...], 0.0)
    y = jnp.sum(h * w3_ref[...], axis=1, keepdims=True)
    out_ref[...] = y + b3_ref[...] + cr_ref[...]


@jax.jit
def _tc_mlp(le, te, cr, w1a, w1b, b1, w2, b2, w3r, b3):
    grid = _B // _BLK
    full = lambda i: (0, 0)
    return pl.pallas_call(
        _mlp_body,
        grid=(grid,),
        in_specs=[
            pl.BlockSpec((_BLK, _D_LINK), lambda i: (i, 0)),
            pl.BlockSpec((_BLK, _D_TIME), lambda i: (i, 0)),
            pl.BlockSpec((_BLK, 1), lambda i: (i, 0)),
            pl.BlockSpec((_D_LINK, 128), full),
            pl.BlockSpec((_D_TIME, 128), full),
            pl.BlockSpec((1, 128), full),
            pl.BlockSpec((128, 64), full),
            pl.BlockSpec((1, 64), full),
            pl.BlockSpec((1, 64), full),
            pl.BlockSpec((1, 1), full),
        ],
        out_specs=pl.BlockSpec((_BLK, 1), lambda i: (i, 0)),
        out_shape=jax.ShapeDtypeStruct((_B, 1), jnp.float32),
    )(le, te, cr, w1a, w1b, b1, w2, b2, w3r, b3)


def kernel(link_idx, time_idx, link_table, time_table, cross_table,
           W1, b1, W2, b2, W3, b3):
    le, te, cr = _sc_gather(
        link_idx.astype(jnp.int32), time_idx.astype(jnp.int32),
        link_table, time_table, cross_table.reshape(-1, 128))
    y = _tc_mlp(
        le, te, cr[:, None],
        W1[:_D_LINK], W1[_D_LINK:], b1[None, :],
        W2, b2[None, :], W3.reshape(1, 64), b3[None, :])
    return y[:, 0]
